# baseline (speedup 1.0000x reference)
import functools

import jax
import jax.numpy as jnp
from jax import lax
from jax.experimental import pallas as pl
from jax.experimental.pallas import tpu as pltpu
from jax.experimental.pallas import tpu_sc as plsc


_MAX_LEN = 8192
_D_MODEL = 1024
_BATCH = 4

_NC = 2
_NS = 16
_NW = _NC * _NS
_ROWS_PER_W = _MAX_LEN // _NW
_CH = 32
_NCH = _ROWS_PER_W // _CH
_NBUF = 3


@functools.partial(
    pl.kernel,
    mesh=plsc.VectorSubcoreMesh(core_axis_name="c", subcore_axis_name="s"),
    out_type=jax.ShapeDtypeStruct((_BATCH, _MAX_LEN, _D_MODEL), jnp.float32),
    scratch_types=[
        pltpu.VMEM((_NBUF, _CH, _D_MODEL), jnp.float32),
    ],
)
def _sc_broadcast(w_hbm, o_hbm, buf):
    wid = lax.axis_index("s") * _NC + lax.axis_index("c")
    base = wid * _ROWS_PER_W

    for k in range(_NCH):
        s = k % _NBUF
        pltpu.sync_copy(w_hbm.at[pl.ds(base + k * _CH, _CH), :], buf.at[s])
        for b in range(_BATCH):
            pltpu.sync_copy(buf.at[s], o_hbm.at[b, pl.ds(base + k * _CH, _CH), :])


def kernel(x, pe_weight):
    del x
    return _sc_broadcast(pe_weight)


# speedup vs baseline: 1.0443x; 1.0443x over previous
import functools

import jax
import jax.numpy as jnp
from jax import lax
from jax.experimental import pallas as pl
from jax.experimental.pallas import tpu as pltpu
from jax.experimental.pallas import tpu_sc as plsc


_MAX_LEN = 8192
_D_MODEL = 1024
_BATCH = 4

_NC = 2
_NS = 16
_NW = _NC * _NS
_ROWS_PER_W = _MAX_LEN // _NW
_CH = 32
_NCH = _ROWS_PER_W // _CH
_NBUF = 2


@functools.partial(
    pl.kernel,
    mesh=plsc.VectorSubcoreMesh(core_axis_name="c", subcore_axis_name="s"),
    out_type=jax.ShapeDtypeStruct((_BATCH, _MAX_LEN, _D_MODEL), jnp.float32),
    scratch_types=[
        pltpu.VMEM((_NBUF, _CH, _D_MODEL), jnp.float32),
        pltpu.SemaphoreType.DMA((_NBUF,)),
        pltpu.SemaphoreType.DMA((_BATCH,)),
    ],
)
def _sc_broadcast(w_hbm, o_hbm, buf, in_sems, out_sems):
    wid = lax.axis_index("s") * _NC + lax.axis_index("c")
    base = wid * _ROWS_PER_W

    def in_copy(k):
        s = k % _NBUF
        return pltpu.make_async_copy(
            w_hbm.at[pl.ds(base + k * _CH, _CH), :], buf.at[s], in_sems.at[s]
        )

    def out_copy(k, b):
        return pltpu.make_async_copy(
            buf.at[k % _NBUF],
            o_hbm.at[b, pl.ds(base + k * _CH, _CH), :],
            out_sems.at[b],
        )

    in_copy(0).start()
    for k in range(_NCH):
        in_copy(k).wait()
        if k + 1 < _NCH:
            in_copy(k + 1).start()
        for b in range(_BATCH):
            out_copy(k, b).start()
        for b in range(_BATCH):
            out_copy(k, b).wait()


def kernel(x, pe_weight):
    del x
    return _sc_broadcast(pe_weight)


# speedup vs baseline: 1.5934x; 1.5258x over previous
import jax
import jax.numpy as jnp
from jax.experimental import pallas as pl
from jax.experimental.pallas import tpu as pltpu


_NCHUNK = 32


def _body(w_hbm, o_hbm, buf, in_sems, out_sems):
    n_rows, _ = w_hbm.shape
    batch = o_hbm.shape[0]
    chunk = n_rows // _NCHUNK

    def in_copy(c):
        sl = pl.ds(c * chunk, chunk)
        return pltpu.make_async_copy(w_hbm.at[sl, :], buf.at[sl, :], in_sems.at[c])

    def out_copy(c, b):
        sl = pl.ds(c * chunk, chunk)
        return pltpu.make_async_copy(buf.at[sl, :], o_hbm.at[b, sl, :], out_sems.at[b])

    for c in range(_NCHUNK):
        in_copy(c).start()
    for c in range(_NCHUNK):
        in_copy(c).wait()
        for b in range(batch):
            out_copy(c, b).start()
    for c in range(_NCHUNK):
        for b in range(batch):
            out_copy(c, b).wait()


def kernel(x, pe_weight):
    batch = x.shape[0]
    max_len, d_model = pe_weight.shape
    return pl.pallas_call(
        _body,
        in_specs=[pl.BlockSpec(memory_space=pl.ANY)],
        out_specs=pl.BlockSpec(memory_space=pl.ANY),
        out_shape=jax.ShapeDtypeStruct((batch, max_len, d_model), pe_weight.dtype),
        scratch_shapes=[
            pltpu.VMEM((max_len, d_model), pe_weight.dtype),
            pltpu.SemaphoreType.DMA((_NCHUNK,)),
            pltpu.SemaphoreType.DMA((4,)),
        ],
    )(pe_weight)
